# Initial kernel scaffold; baseline (speedup 1.0000x reference)
import jax
import jax.numpy as jnp
from jax.experimental import pallas as pl


def kernel(x_nchw, w_depthwise, w_pointwise):
    raise NotImplementedError("write your pallas kernel here")


# speedup vs baseline: 1.0746x; 1.0746x over previous
import jax
import jax.numpy as jnp
from jax import lax
from jax.experimental import pallas as pl
from jax.experimental.pallas import tpu as pltpu


def _make_body(bt, Cin, Cout, H, W, HWp, K, pad):
    def body(x_ref, wd_ref, wp_ref, o_ref):

        pos = lax.broadcasted_iota(jnp.int32, (1, HWp), 1)
        row = pos // W
        col = pos % W
        col_mask = {dw: (col + dw >= 0) & (col + dw < W)
                    for dw in range(-pad, pad + 1) if dw != 0}
        row_mask = {dh: (row + dh >= 0) & (row + dh < H)
                    for dh in range(-pad, pad + 1) if dh != 0}
        wp = wp_ref[...]

        for bi in range(bt):
            x = x_ref[bi]
            xs = {0: x}
            for dw, m in col_mask.items():
                xs[dw] = jnp.where(m, jnp.roll(x, -dw, axis=1), 0.0)
            y = None
            for dh in range(-pad, pad + 1):
                s = None
                for dw in range(-pad, pad + 1):
                    t = (dh + pad) * K + (dw + pad)
                    term = xs[dw] * wd_ref[:, t:t + 1]
                    s = term if s is None else s + term
                if dh != 0:
                    s = jnp.where(row_mask[dh], jnp.roll(s, -dh * W, axis=1),
                                  0.0)
                y = s if y is None else y + s
            acc = jnp.dot(wp, y.astype(jnp.bfloat16),
                          preferred_element_type=jnp.float32)
            o_ref[bi] = acc

    return body


def kernel(x_nchw, w_depthwise, w_pointwise):
    N, Cin, H, W = x_nchw.shape
    K = w_depthwise.shape[-1]
    Cout = w_pointwise.shape[0]
    pad = (K - 1) // 2
    HW = H * W
    HWp = ((HW + 127) // 128) * 128

    x_flat = x_nchw.reshape(N, Cin, HW)
    if HWp != HW:
        x_flat = jnp.pad(x_flat, ((0, 0), (0, 0), (0, HWp - HW)))

    wd = w_depthwise[:, 0, :, :].reshape(Cin, K * K).astype(jnp.float32)
    wp = w_pointwise[:, :, 0, 0].astype(jnp.bfloat16)

    bt = 2 if N % 2 == 0 else 1

    flops = 2 * N * HWp * Cin * Cout + 2 * N * HWp * Cin * K * K
    bytes_accessed = (N * Cin * HWp * 4 + N * Cout * HWp * 4
                      + Cin * K * K * 4 + Cout * Cin * 2)

    out_flat = pl.pallas_call(
        _make_body(bt, Cin, Cout, H, W, HWp, K, pad),
        out_shape=jax.ShapeDtypeStruct((N, Cout, HWp), x_nchw.dtype),
        grid_spec=pltpu.PrefetchScalarGridSpec(
            num_scalar_prefetch=0,
            grid=(N // bt,),
            in_specs=[
                pl.BlockSpec((bt, Cin, HWp), lambda b: (b, 0, 0)),
                pl.BlockSpec((Cin, K * K), lambda b: (0, 0)),
                pl.BlockSpec((Cout, Cin), lambda b: (0, 0)),
            ],
            out_specs=pl.BlockSpec((bt, Cout, HWp), lambda b: (b, 0, 0)),
        ),
        compiler_params=pltpu.CompilerParams(
            dimension_semantics=("parallel",),
            vmem_limit_bytes=64 * 1024 * 1024),
        cost_estimate=pl.CostEstimate(
            flops=flops, transcendentals=0, bytes_accessed=bytes_accessed),
    )(x_flat, wd, wp)

    out_flat = out_flat[:, :, :HW] if HWp != HW else out_flat
    return out_flat.reshape(N, Cout, H, W)


# speedup vs baseline: 3.4725x; 3.2315x over previous
import jax
import jax.numpy as jnp
from jax import lax
from jax.experimental import pallas as pl
from jax.experimental.pallas import tpu as pltpu


def _make_body(bt, Cin, Cout, H, W, K, pad):
    HW = H * W

    def body(x_ref, wd_ref, wp_ref, o_ref):

        pos = lax.broadcasted_iota(jnp.int32, (HW, 1), 0)
        row = pos // W
        col = pos % W
        col_mask = {dw: (col + dw >= 0) & (col + dw < W)
                    for dw in range(-pad, pad + 1) if dw != 0}
        row_mask = {dh: (row + dh >= 0) & (row + dh < H)
                    for dh in range(-pad, pad + 1) if dh != 0}
        wp = wp_ref[...]

        for bi in range(bt):
            x = x_ref[bi]
            xs = {0: x}
            for dw, m in col_mask.items():
                xs[dw] = jnp.where(m, jnp.roll(x, -dw, axis=0), 0.0)
            y = None
            for dh in range(-pad, pad + 1):
                s = None
                for dw in range(-pad, pad + 1):
                    t = (dh + pad) * K + (dw + pad)
                    term = xs[dw] * wd_ref[t:t + 1, :]
                    s = term if s is None else s + term
                if dh != 0:
                    s = jnp.where(row_mask[dh], jnp.roll(s, -dh * W, axis=0),
                                  0.0)
                y = s if y is None else y + s
            acc = jnp.dot(y.astype(jnp.bfloat16), wp,
                          preferred_element_type=jnp.float32)
            o_ref[bi] = acc

    return body


def kernel(x_nchw, w_depthwise, w_pointwise):
    N, Cin, H, W = x_nchw.shape
    K = w_depthwise.shape[-1]
    Cout = w_pointwise.shape[0]
    pad = (K - 1) // 2
    HW = H * W

    x_pc = jnp.transpose(x_nchw, (0, 2, 3, 1)).reshape(N, HW, Cin)

    wd = jnp.transpose(w_depthwise[:, 0, :, :], (1, 2, 0)).reshape(K * K, Cin)
    wd = wd.astype(jnp.float32)
    wp = jnp.transpose(w_pointwise[:, :, 0, 0], (1, 0)).astype(jnp.bfloat16)

    bt = 2 if N % 2 == 0 else 1

    flops = 2 * N * HW * Cin * Cout + 2 * N * HW * Cin * K * K
    bytes_accessed = (N * Cin * HW * 4 + N * Cout * HW * 4
                      + Cin * K * K * 4 + Cout * Cin * 2)

    out_pc = pl.pallas_call(
        _make_body(bt, Cin, Cout, H, W, K, pad),
        out_shape=jax.ShapeDtypeStruct((N, HW, Cout), x_nchw.dtype),
        grid_spec=pltpu.PrefetchScalarGridSpec(
            num_scalar_prefetch=0,
            grid=(N // bt,),
            in_specs=[
                pl.BlockSpec((bt, HW, Cin), lambda b: (b, 0, 0)),
                pl.BlockSpec((K * K, Cin), lambda b: (0, 0)),
                pl.BlockSpec((Cin, Cout), lambda b: (0, 0)),
            ],
            out_specs=pl.BlockSpec((bt, HW, Cout), lambda b: (b, 0, 0)),
        ),
        compiler_params=pltpu.CompilerParams(
            dimension_semantics=("parallel",),
            vmem_limit_bytes=64 * 1024 * 1024),
        cost_estimate=pl.CostEstimate(
            flops=flops, transcendentals=0, bytes_accessed=bytes_accessed),
    )(x_pc, wd, wp)

    return jnp.transpose(out_pc.reshape(N, H, W, Cout), (0, 3, 1, 2))
